# baseline (speedup 1.0000x reference)
import functools

import jax
import jax.numpy as jnp
from jax import lax
from jax.experimental import pallas as pl
from jax.experimental.pallas import tpu as pltpu
from jax.experimental.pallas import tpu_sc as plsc

_LANES = 16
_TW = 128
_GRP = 8
_NCHUNK = 2


@functools.lru_cache(maxsize=None)
def _make_sc_gather(nz, nx, batch):
    info = plsc.get_sparse_core_info()
    nc, ns = info.num_cores, info.num_subcores
    nw = nc * ns
    assert batch % (8 * nw) == 0
    b_per_w = batch // nw
    n_groups = b_per_w // _GRP
    assert n_groups % 2 == 0
    mesh = plsc.VectorSubcoreMesh(core_axis_name="c", subcore_axis_name="s")

    @functools.partial(
        pl.kernel,
        mesh=mesh,
        out_type=jax.ShapeDtypeStruct((batch, nz), jnp.float32),
        scratch_types=[
            pltpu.VMEM((b_per_w + _LANES,), jnp.int32),
            pltpu.VMEM((b_per_w + _LANES,), jnp.int32),
            pltpu.VMEM((_GRP, nz, _TW), jnp.float32),
            pltpu.VMEM((_GRP, nz, _TW), jnp.float32),
            pltpu.VMEM((b_per_w, nz), jnp.float32),
            pltpu.SemaphoreType.DMA,
            pltpu.SemaphoreType.DMA,
        ],
        compiler_params=pltpu.CompilerParams(needs_layout_passes=False),
    )
    def sc_gather(x_hbm, t_hbm, ft2_hbm, out_hbm, xv, tv, bufa, bufb, rows,
                  sema, semb):
        wid = lax.axis_index("s") * nc + lax.axis_index("c")
        base = wid * b_per_w
        pltpu.sync_copy(x_hbm.at[pl.ds(base, b_per_w)],
                        xv.at[pl.ds(0, b_per_w)])
        pltpu.sync_copy(t_hbm.at[pl.ds(base, b_per_w)],
                        tv.at[pl.ds(0, b_per_w)])

        iota = lax.iota(jnp.int32, _LANES)

        def issue(gbase, buf, sem):
            xvec = xv[pl.ds(gbase, _LANES)] & ~(_TW - 1)
            tvec = tv[pl.ds(gbase, _LANES)] * nz
            for s in range(_GRP):
                src = ft2_hbm.at[pl.ds(pl.multiple_of(tvec[s], 8), nz),
                                 pl.ds(pl.multiple_of(xvec[s], _TW), _TW)]
                pltpu.async_copy(src, buf.at[s], sem)

        def drain(buf, sem):
            for s in range(_GRP):
                pltpu.make_async_copy(
                    ft2_hbm.at[pl.ds(0, nz), pl.ds(0, _TW)],
                    buf.at[s], sem).wait()

        def extract(gbase, buf):
            xvec = xv[pl.ds(gbase, _LANES)] & (_TW - 1)
            for s in range(_GRP):
                lane = jnp.full((_LANES,), xvec[s], jnp.int32)
                for h in range(nz // _LANES):
                    r = plsc.load_gather(buf.at[s], [iota + h * _LANES, lane])
                    rows[gbase + s, pl.ds(h * _LANES, _LANES)] = r

        issue(0, bufa, sema)

        def pair(p, _):
            ga = p * (2 * _GRP)
            issue(ga + _GRP, bufb, semb)
            drain(bufa, sema)
            extract(ga, bufa)
            nxt = jnp.minimum(ga + 2 * _GRP, b_per_w - _GRP)
            issue(nxt, bufa, sema)
            drain(bufb, semb)
            extract(ga + _GRP, bufb)
            return 0

        lax.fori_loop(0, n_groups // 2, pair, 0)
        drain(bufa, sema)

        pltpu.sync_copy(rows, out_hbm.at[pl.ds(base, b_per_w)])

    return sc_gather


def _decode_body(z_ref, w_ref, o_ref):
    z = jnp.maximum(z_ref[...], 0.0)
    o_ref[...] = lax.dot_general(
        z, w_ref[...], (((1,), (1,)), ((), ())),
        preferred_element_type=jnp.float32,
    )


def _decode_acc_body(z_ref, w_ref, acc_ref, o_ref):
    del acc_ref
    _decode_body(z_ref, w_ref, o_ref)


_BLK = 1024


def _decode_into(acc, z, W_dec, off, batch, nz, nd):
    nblk = z.shape[0] // _BLK
    ob = off // _BLK
    specs = [
        pl.BlockSpec((_BLK, nz), lambda i: (i, 0)),
        pl.BlockSpec((nd, nz), lambda i: (0, 0)),
    ]
    args = [z, W_dec]
    aliases = {}
    body = _decode_body
    if acc is not None:
        specs.append(pl.BlockSpec(memory_space=pl.ANY))
        args.append(acc)
        aliases = {2: 0}
        body = _decode_acc_body
    return pl.pallas_call(
        body,
        grid=(nblk,),
        in_specs=specs,
        out_specs=pl.BlockSpec((_BLK, nd), lambda i: (i + ob, 0)),
        out_shape=jax.ShapeDtypeStruct((batch, nd), jnp.float32),
        input_output_aliases=aliases,
    )(*args)


def kernel(x_idx, t_idx, factors, W_dec):
    nx, nt, nz = factors.shape
    nd = W_dec.shape[0]
    batch = x_idx.shape[0]
    ft2 = jnp.transpose(factors, (1, 2, 0)).reshape(nt * nz, nx)
    xi = x_idx.astype(jnp.int32)
    ti = t_idx.astype(jnp.int32)

    chunks = (batch - batch // 16, batch // 16)
    out = None
    off = 0
    for ch in chunks:
        z = _make_sc_gather(nz, nx, ch)(
            lax.dynamic_slice_in_dim(xi, off, ch),
            lax.dynamic_slice_in_dim(ti, off, ch), ft2)
        out = _decode_into(out, z, W_dec, off, batch, nz, nd)
        off += ch
    return out


# speedup vs baseline: 1.0083x; 1.0083x over previous
import functools

import jax
import jax.numpy as jnp
from jax import lax
from jax.experimental import pallas as pl
from jax.experimental.pallas import tpu as pltpu
from jax.experimental.pallas import tpu_sc as plsc

_LANES = 16
_TW = 128
_GRP = 8
_NCHUNK = 2


@functools.lru_cache(maxsize=None)
def _make_sc_gather(nz, nx, batch):
    info = plsc.get_sparse_core_info()
    nc, ns = info.num_cores, info.num_subcores
    nw = nc * ns
    assert batch % (8 * nw) == 0
    b_per_w = batch // nw
    n_groups = b_per_w // _GRP
    assert n_groups % 2 == 0
    mesh = plsc.VectorSubcoreMesh(core_axis_name="c", subcore_axis_name="s")

    @functools.partial(
        pl.kernel,
        mesh=mesh,
        out_type=jax.ShapeDtypeStruct((batch, nz), jnp.float32),
        scratch_types=[
            pltpu.VMEM((b_per_w + _LANES,), jnp.int32),
            pltpu.VMEM((b_per_w + _LANES,), jnp.int32),
            pltpu.VMEM((_GRP, nz, _TW), jnp.float32),
            pltpu.VMEM((_GRP, nz, _TW), jnp.float32),
            pltpu.VMEM((b_per_w, nz), jnp.float32),
            pltpu.SemaphoreType.DMA,
            pltpu.SemaphoreType.DMA,
        ],
        compiler_params=pltpu.CompilerParams(needs_layout_passes=False),
    )
    def sc_gather(x_hbm, t_hbm, ft2_hbm, out_hbm, xv, tv, bufa, bufb, rows,
                  sema, semb):
        wid = lax.axis_index("s") * nc + lax.axis_index("c")
        base = wid * b_per_w
        cx = pltpu.async_copy(x_hbm.at[pl.ds(base, b_per_w)],
                              xv.at[pl.ds(0, b_per_w)], sema)
        ct = pltpu.async_copy(t_hbm.at[pl.ds(base, b_per_w)],
                              tv.at[pl.ds(0, b_per_w)], semb)
        cx.wait()
        ct.wait()

        iota = lax.iota(jnp.int32, _LANES)

        def issue(gbase, buf, sem):
            xvec = xv[pl.ds(gbase, _LANES)] & ~(_TW - 1)
            tvec = tv[pl.ds(gbase, _LANES)] * nz
            for s in range(_GRP):
                src = ft2_hbm.at[pl.ds(pl.multiple_of(tvec[s], 8), nz),
                                 pl.ds(pl.multiple_of(xvec[s], _TW), _TW)]
                pltpu.async_copy(src, buf.at[s], sem)

        def drain(buf, sem):
            for s in range(_GRP):
                pltpu.make_async_copy(
                    ft2_hbm.at[pl.ds(0, nz), pl.ds(0, _TW)],
                    buf.at[s], sem).wait()

        def extract(gbase, buf):
            xvec = xv[pl.ds(gbase, _LANES)] & (_TW - 1)
            for s in range(_GRP):
                lane = jnp.full((_LANES,), xvec[s], jnp.int32)
                for h in range(nz // _LANES):
                    r = plsc.load_gather(buf.at[s], [iota + h * _LANES, lane])
                    rows[gbase + s, pl.ds(h * _LANES, _LANES)] = r

        issue(0, bufa, sema)

        def pair(p, _):
            ga = p * (2 * _GRP)
            issue(ga + _GRP, bufb, semb)
            drain(bufa, sema)
            extract(ga, bufa)
            nxt = jnp.minimum(ga + 2 * _GRP, b_per_w - _GRP)
            issue(nxt, bufa, sema)
            drain(bufb, semb)
            extract(ga + _GRP, bufb)
            return 0

        lax.fori_loop(0, n_groups // 2, pair, 0)
        drain(bufa, sema)

        pltpu.sync_copy(rows, out_hbm.at[pl.ds(base, b_per_w)])

    return sc_gather


def _decode_body(z_ref, w_ref, o_ref):
    z = jnp.maximum(z_ref[...], 0.0)
    o_ref[...] = lax.dot_general(
        z, w_ref[...], (((1,), (1,)), ((), ())),
        preferred_element_type=jnp.float32,
    )


def _decode_acc_body(z_ref, w_ref, acc_ref, o_ref):
    del acc_ref
    _decode_body(z_ref, w_ref, o_ref)


_BLK = 2048


def _decode_into(acc, z, W_dec, off, batch, nz, nd):
    nblk = z.shape[0] // _BLK
    ob = off // _BLK
    specs = [
        pl.BlockSpec((_BLK, nz), lambda i: (i, 0)),
        pl.BlockSpec((nd, nz), lambda i: (0, 0)),
    ]
    args = [z, W_dec]
    aliases = {}
    body = _decode_body
    if acc is not None:
        specs.append(pl.BlockSpec(memory_space=pl.ANY))
        args.append(acc)
        aliases = {2: 0}
        body = _decode_acc_body
    return pl.pallas_call(
        body,
        grid=(nblk,),
        in_specs=specs,
        out_specs=pl.BlockSpec((_BLK, nd), lambda i: (i + ob, 0)),
        out_shape=jax.ShapeDtypeStruct((batch, nd), jnp.float32),
        input_output_aliases=aliases,
    )(*args)


def kernel(x_idx, t_idx, factors, W_dec):
    nx, nt, nz = factors.shape
    nd = W_dec.shape[0]
    batch = x_idx.shape[0]
    ft2 = jnp.transpose(factors, (1, 2, 0)).reshape(nt * nz, nx)
    xi = x_idx.astype(jnp.int32)
    ti = t_idx.astype(jnp.int32)

    chunks = (batch - batch // 8, batch // 8)
    out = None
    off = 0
    for ch in chunks:
        z = _make_sc_gather(nz, nx, ch)(
            lax.dynamic_slice_in_dim(xi, off, ch),
            lax.dynamic_slice_in_dim(ti, off, ch), ft2)
        out = _decode_into(out, z, W_dec, off, batch, nz, nd)
        off += ch
    return out
